# baseline (speedup 1.0000x reference)
import functools

import jax
import jax.numpy as jnp
from jax import lax
from jax.experimental import pallas as pl
from jax.experimental.pallas import tpu as pltpu
from jax.experimental.pallas import tpu_sc as plsc

S = 12
SS = S * S
C = 96
H = 512
W = 512
NW = 32
RPW = 32
RPAD = NW * RPW
NCHUNK = SS // 16


def _splat(vec, lane):
    idx = jnp.full((16, 1), lane, jnp.int32)
    dnums = lax.GatherDimensionNumbers(
        offset_dims=(), collapsed_slice_dims=(0,), start_index_map=(0,))
    return lax.gather(vec, idx, dnums, (1,),
                      mode=lax.GatherScatterMode.PROMISE_IN_BOUNDS)


def _sc_body(xt_hbm, rois_hbm, out_hbm, roisv, idxv, wbuf, gbuf, obuf, sem):
    wid = lax.axis_index("s") * 2 + lax.axis_index("c")
    rbase = wid * RPW
    pltpu.sync_copy(rois_hbm.at[pl.ds(rbase, RPW)], roisv)

    lanes = lax.iota(jnp.int32, 16)

    def roi_body(i, carry):
        row = roisv[i, :]
        x1v = _splat(row, 1)
        y1v = _splat(row, 2)
        x2v = _splat(row, 3)
        y2v = _splat(row, 4)
        bwv = (x2v - x1v) / float(S)
        bhv = (y2v - y1v) / float(S)

        def chunk_body(j, carry2):
            s_i = lanes + j * 16
            sy = lax.div(s_i, S)
            sx = s_i - sy * S
            Yv = y1v + (sy.astype(jnp.float32) + 0.5) * bhv
            Xv = x1v + (sx.astype(jnp.float32) + 0.5) * bwv
            y0i = Yv.astype(jnp.int32)
            x0i = Xv.astype(jnp.int32)
            ly = Yv - y0i.astype(jnp.float32)
            lx = Xv - x0i.astype(jnp.float32)
            hy = 1.0 - ly
            hx = 1.0 - lx
            y0c = jnp.minimum(jnp.maximum(y0i, 0), H - 1)
            x0c = jnp.minimum(jnp.maximum(x0i, 0), W - 1)
            y1c = jnp.minimum(y0c + 1, H - 1)
            x1c = jnp.minimum(x0c + 1, W - 1)
            b = j * 64
            idxv[pl.ds(b, 16)] = y0c * W + x0c
            idxv[pl.ds(b + 16, 16)] = y0c * W + x1c
            idxv[pl.ds(b + 32, 16)] = y1c * W + x0c
            idxv[pl.ds(b + 48, 16)] = y1c * W + x1c
            wbuf[0, pl.ds(j * 16, 16)] = hy * hx
            wbuf[1, pl.ds(j * 16, 16)] = hy * lx
            wbuf[2, pl.ds(j * 16, 16)] = ly * hx
            wbuf[3, pl.ds(j * 16, 16)] = ly * lx
            return carry2

        lax.fori_loop(0, NCHUNK, chunk_body, 0)

        pltpu.async_copy(xt_hbm.at[idxv], gbuf, sem).wait()

        def blend_body(j, carry2):
            w00 = wbuf[0, pl.ds(j * 16, 16)]
            w01 = wbuf[1, pl.ds(j * 16, 16)]
            w10 = wbuf[2, pl.ds(j * 16, 16)]
            w11 = wbuf[3, pl.ds(j * 16, 16)]
            r00 = lanes + j * 64
            r01 = r00 + 16
            r10 = r00 + 32
            r11 = r00 + 48

            def chan_body(c, carry3):
                cv = jnp.broadcast_to(c, (16,))
                v00 = plsc.load_gather(gbuf, [r00, cv])
                v01 = plsc.load_gather(gbuf, [r01, cv])
                v10 = plsc.load_gather(gbuf, [r10, cv])
                v11 = plsc.load_gather(gbuf, [r11, cv])
                acc = v00 * w00 + v01 * w01 + v10 * w10 + v11 * w11
                obuf[c, pl.ds(j * 16, 16)] = acc
                return carry3

            lax.fori_loop(0, C, chan_body, 0)
            return carry2

        lax.fori_loop(0, NCHUNK, blend_body, 0)

        pltpu.sync_copy(obuf, out_hbm.at[rbase + i])
        return carry

    lax.fori_loop(0, RPW, roi_body, 0)


@jax.jit
def _roialign_sc(xt, rois_p):
    mesh = plsc.VectorSubcoreMesh(core_axis_name="c", subcore_axis_name="s")
    kfn = functools.partial(
        pl.kernel,
        mesh=mesh,
        out_type=jax.ShapeDtypeStruct((RPAD, C, SS), jnp.float32),
        scratch_types=[
            pltpu.VMEM((RPW, 16), jnp.float32),
            pltpu.VMEM((4 * SS,), jnp.int32),
            pltpu.VMEM((4, SS), jnp.float32),
            pltpu.VMEM((4 * SS, C), jnp.float32),
            pltpu.VMEM((C, SS), jnp.float32),
            pltpu.SemaphoreType.DMA,
        ],
        compiler_params=pltpu.CompilerParams(
            needs_layout_passes=False, use_tc_tiling_on_sc=False),
    )(_sc_body)
    return kfn(xt, rois_p)


def kernel(x, rois):
    N, c, h, w = x.shape
    xt = jnp.transpose(x[0], (1, 2, 0)).reshape(H * W, C)
    rois_p = jnp.pad(rois, ((0, RPAD - rois.shape[0]), (0, 11)))
    out = _roialign_sc(xt, rois_p)
    return out[: rois.shape[0]].reshape(rois.shape[0], C, S, S)
